# Initial kernel scaffold; baseline (speedup 1.0000x reference)
import jax
import jax.numpy as jnp
from jax.experimental import pallas as pl


def kernel(x, W0, b0, W1, b1, W2, b2, Wc1, bc1, Wc2, bc2, Wr1, br1, Wr2, br2, edge_index, batch):
    raise NotImplementedError("write your pallas kernel here")


# speedup vs baseline: 12.1338x; 12.1338x over previous
import functools

import jax
import jax.numpy as jnp
from jax import lax
from jax.experimental import pallas as pl
from jax.experimental.pallas import tpu as pltpu
from jax.experimental.pallas import tpu_sc as plsc

N = 50000
E = 800000
G = 500
HID = 64

NPAD = 51200
EPAD = 819200
GPAD = 512
TRASH = NPAD - 1

ROWS_PER_TILE = NPAD // 16
DEG_CHUNKS = EPAD // (32 * 128)
LAY_CHUNKS = EPAD // (16 * 128)
POOL_ROWS = 416

_R = 512
_GRID = NPAD // _R

_SC_MESH = plsc.VectorSubcoreMesh(core_axis_name="c", subcore_axis_name="s")


@functools.partial(
    pl.kernel,
    out_type=jax.ShapeDtypeStruct((2 * NPAD, 8), jnp.float32),
    mesh=_SC_MESH,
    compiler_params=pltpu.CompilerParams(use_tc_tiling_on_sc=False),
    scratch_types=[
        pltpu.VMEM((DEG_CHUNKS, 128), jnp.int32),
        pltpu.VMEM((128, 8), jnp.float32),
        pltpu.VMEM_SHARED((NPAD, 8), jnp.float32),
    ],
)
def _sc_deg(didx_hbm, ones_hbm, zeros_hbm, out_hbm, didx_v, ones_v, accum):
    c = lax.axis_index("c")
    s = lax.axis_index("s")
    w = s * 2 + c
    stripe = pl.multiple_of(s * ROWS_PER_TILE, 8)
    out_off = pl.multiple_of(c * NPAD + s * ROWS_PER_TILE, 8)
    pltpu.sync_copy(didx_hbm.at[pl.ds(pl.multiple_of(w * DEG_CHUNKS, 8), DEG_CHUNKS)],
                    didx_v)
    pltpu.sync_copy(ones_hbm, ones_v)
    pltpu.sync_copy(zeros_hbm, accum.at[pl.ds(stripe, ROWS_PER_TILE)])
    plsc.subcore_barrier()

    def body(j, carry):
        pltpu.sync_copy(ones_v, accum.at[didx_v.at[j]], add=True)
        return carry

    lax.fori_loop(0, DEG_CHUNKS, body, 0)
    plsc.subcore_barrier()
    pltpu.sync_copy(
        accum.at[pl.ds(stripe, ROWS_PER_TILE)],
        out_hbm.at[pl.ds(out_off, ROWS_PER_TILE)],
    )


_KBUF = 4
_IBLK = 40
_IBLK_N = LAY_CHUNKS // _IBLK


@functools.partial(
    pl.kernel,
    out_type=jax.ShapeDtypeStruct((2 * NPAD, 32), jnp.float32),
    mesh=_SC_MESH,
    compiler_params=pltpu.CompilerParams(use_tc_tiling_on_sc=False),
    scratch_types=[
        pltpu.VMEM((_IBLK, 128), jnp.int32),
        pltpu.VMEM((_IBLK, 128), jnp.int32),
        pltpu.VMEM((_KBUF, 128, 32), jnp.float32),
        pltpu.VMEM_SHARED((NPAD, 32), jnp.float32),
        pltpu.SemaphoreType.DMA,
    ],
)
def _sc_layer(yflat_hbm, gidx_hbm, didx_hbm, zeros_hbm, out_hbm,
              gidx_v, didx_v, rows_v, accum, gsem):
    c = lax.axis_index("c")
    s = lax.axis_index("s")
    stripe = pl.multiple_of(s * ROWS_PER_TILE, 8)
    out_off = pl.multiple_of(c * NPAD + s * ROWS_PER_TILE, 8)
    idx_base = pl.multiple_of((c * 16 + s) * LAY_CHUNKS, 8)
    didx_base = pl.multiple_of(s * LAY_CHUNKS, 8)
    pltpu.sync_copy(zeros_hbm, accum.at[pl.ds(stripe, ROWS_PER_TILE)])
    plsc.subcore_barrier()

    def block_body(blk, carry):
        boff = pl.multiple_of(blk * _IBLK, 8)
        pltpu.sync_copy(gidx_hbm.at[pl.ds(idx_base + boff, _IBLK)], gidx_v)
        pltpu.sync_copy(didx_hbm.at[pl.ds(didx_base + boff, _IBLK)], didx_v)

        def round_body(g, carry2):
            base = g * _KBUF
            cps = [
                pltpu.async_copy(yflat_hbm.at[gidx_v.at[base + k]],
                                 rows_v.at[k], gsem)
                for k in range(_KBUF)
            ]
            for k in range(_KBUF):
                cps[k].wait()
            for k in range(_KBUF):
                pltpu.sync_copy(rows_v.at[k], accum.at[didx_v.at[base + k]],
                                add=True)
            return carry2

        lax.fori_loop(0, _IBLK // _KBUF, round_body, 0)
        return carry

    lax.fori_loop(0, _IBLK_N, block_body, 0)
    plsc.subcore_barrier()
    pltpu.sync_copy(
        accum.at[pl.ds(stripe, ROWS_PER_TILE)],
        out_hbm.at[pl.ds(out_off, ROWS_PER_TILE)],
    )


@functools.partial(
    pl.kernel,
    out_type=(
        jax.ShapeDtypeStruct((2 * GPAD, HID), jnp.float32),
        jax.ShapeDtypeStruct((2 * GPAD, 8), jnp.float32),
    ),
    mesh=_SC_MESH,
    compiler_params=pltpu.CompilerParams(use_tc_tiling_on_sc=False),
    scratch_types=[
        pltpu.VMEM((24, 128), jnp.int32),
        pltpu.VMEM((128, HID), jnp.float32),
        pltpu.VMEM((128, 8), jnp.float32),
        pltpu.VMEM_SHARED((GPAD, HID), jnp.float32),
        pltpu.VMEM_SHARED((GPAD, 8), jnp.float32),
    ],
)
def _sc_pool(h3_hbm, bidx_hbm, ones_hbm, zsum_hbm, zcnt_hbm,
             sums_hbm, cnts_hbm, bidx_v, rows_v, ones_v, accum, cacc):
    c = lax.axis_index("c")
    s = lax.axis_index("s")
    nch = jnp.where(s < 8, 13, 12)
    base = c * 200 + s * 12 + jnp.minimum(s, 8)
    base_al = pl.multiple_of((base // 8) * 8, 8)
    off = base - base_al
    pltpu.sync_copy(bidx_hbm.at[pl.ds(base_al, 24)], bidx_v)
    pltpu.sync_copy(ones_hbm, ones_v)

    @pl.when(s == 0)
    def _():
        pltpu.sync_copy(zsum_hbm, accum)
        pltpu.sync_copy(zcnt_hbm, cacc)

    plsc.subcore_barrier()

    def body(j, carry):
        ch = pl.multiple_of((base + j) * 128, 128)
        pltpu.sync_copy(h3_hbm.at[pl.ds(ch, 128)], rows_v)
        pltpu.sync_copy(rows_v, accum.at[bidx_v.at[off + j]], add=True)
        pltpu.sync_copy(ones_v, cacc.at[bidx_v.at[off + j]], add=True)
        return carry

    lax.fori_loop(0, nch, body, 0)
    plsc.subcore_barrier()
    pltpu.sync_copy(accum.at[pl.ds(pl.multiple_of(s * 32, 8), 32)],
                    sums_hbm.at[pl.ds(pl.multiple_of(c * GPAD + s * 32, 8), 32)])
    pltpu.sync_copy(cacc.at[pl.ds(pl.multiple_of(s * 32, 8), 32)],
                    cnts_hbm.at[pl.ds(pl.multiple_of(c * GPAD + s * 32, 8), 32)])


def _tc_prep0(degp, x, w0):
    def body(d0_ref, d1_ref, x_ref, w0_ref, dinv_ref, xw_ref, y_ref):
        deg = d0_ref[:, 0:1] + d1_ref[:, 0:1] + 1.0
        dinv = lax.rsqrt(deg)
        xw = jnp.dot(x_ref[...], w0_ref[...], preferred_element_type=jnp.float32)
        dinv_ref[...] = dinv
        xw_ref[...] = xw
        y_ref[...] = dinv * xw

    return pl.pallas_call(
        body,
        grid=(_GRID,),
        in_specs=[
            pl.BlockSpec((_R, 8), lambda i: (i, 0)),
            pl.BlockSpec((_R, 8), lambda i: (i + _GRID, 0)),
            pl.BlockSpec((_R, 3), lambda i: (i, 0)),
            pl.BlockSpec((3, HID), lambda i: (0, 0)),
        ],
        out_specs=[
            pl.BlockSpec((_R, 1), lambda i: (i, 0)),
            pl.BlockSpec((_R, HID), lambda i: (i, 0)),
            pl.BlockSpec((_R, HID), lambda i: (i, 0)),
        ],
        out_shape=[
            jax.ShapeDtypeStruct((NPAD, 1), jnp.float32),
            jax.ShapeDtypeStruct((NPAD, HID), jnp.float32),
            jax.ShapeDtypeStruct((NPAD, HID), jnp.float32),
        ],
    )(degp, degp, x, w0)


def _tc_mid(scat, xwp, dinv, b, w):
    def body(s0_ref, s1_ref, xwp_ref, dinv_ref, b_ref, w_ref, y_ref, xw_ref):
        dv = dinv_ref[...]
        sc = jnp.concatenate([s0_ref[...], s1_ref[...]], axis=1)
        h = jnp.maximum(dv * sc + dv * dv * xwp_ref[...] + b_ref[...], 0.0)
        xwn = jnp.dot(h, w_ref[...], preferred_element_type=jnp.float32)
        xw_ref[...] = xwn
        y_ref[...] = dv * xwn

    return pl.pallas_call(
        body,
        grid=(_GRID,),
        in_specs=[
            pl.BlockSpec((_R, 32), lambda i: (i, 0)),
            pl.BlockSpec((_R, 32), lambda i: (i + _GRID, 0)),
            pl.BlockSpec((_R, HID), lambda i: (i, 0)),
            pl.BlockSpec((_R, 1), lambda i: (i, 0)),
            pl.BlockSpec((1, HID), lambda i: (0, 0)),
            pl.BlockSpec((HID, HID), lambda i: (0, 0)),
        ],
        out_specs=[
            pl.BlockSpec((_R, HID), lambda i: (i, 0)),
            pl.BlockSpec((_R, HID), lambda i: (i, 0)),
        ],
        out_shape=[
            jax.ShapeDtypeStruct((NPAD, HID), jnp.float32),
            jax.ShapeDtypeStruct((NPAD, HID), jnp.float32),
        ],
    )(scat, scat, xwp, dinv, b, w)


def _tc_head(sums, cnts, wc1, bc1, wc2, bc2, wr1, br1, wr2, br2):
    def body(p_ref, c_ref, wc1_ref, bc1_ref, wc2_ref, bc2_ref,
             wr1_ref, br1_ref, wr2_ref, br2_ref, cls_ref, reg_ref):
        ssum = p_ref[0:GPAD, :] + p_ref[GPAD:2 * GPAD, :]
        cnt = c_ref[0:GPAD, 0:1] + c_ref[GPAD:2 * GPAD, 0:1]
        pooled = ssum / jnp.maximum(cnt, 1.0)
        hc = jnp.maximum(
            jnp.dot(pooled, wc1_ref[...], preferred_element_type=jnp.float32)
            + bc1_ref[...], 0.0)
        cls_ref[...] = (jnp.dot(hc, wc2_ref[...], preferred_element_type=jnp.float32)
                        + bc2_ref[...])
        hr = jnp.maximum(
            jnp.dot(pooled, wr1_ref[...], preferred_element_type=jnp.float32)
            + br1_ref[...], 0.0)
        reg_ref[...] = (jnp.dot(hr, wr2_ref[...], preferred_element_type=jnp.float32)
                        + br2_ref[...])

    return pl.pallas_call(
        body,
        out_shape=[
            jax.ShapeDtypeStruct((GPAD, 10), jnp.float32),
            jax.ShapeDtypeStruct((GPAD, 1), jnp.float32),
        ],
    )(sums, cnts, wc1, bc1, wc2, bc2, wr1, br1, wr2, br2)


def kernel(x, W0, b0, W1, b1, W2, b2, Wc1, bc1, Wc2, bc2, Wr1, br1, Wr2, br2,
           edge_index, batch):
    f32 = jnp.float32
    src = edge_index[0]
    dst = edge_index[1]
    pad_e = jnp.full((EPAD - E,), TRASH, jnp.int32)
    src_p = jnp.concatenate([src, pad_e])
    dst_p = jnp.concatenate([dst, pad_e])
    x_p = jnp.pad(x, ((0, NPAD - N), (0, 0)))
    batch_p = jnp.concatenate(
        [batch, jnp.full((POOL_ROWS * 128 - N,), GPAD - 1, jnp.int32)])

    didx_deg = dst_p.reshape(32 * DEG_CHUNKS, 128)
    didx_lay = dst_p.reshape(16 * LAY_CHUNKS, 128)
    gidx = ((2 * src_p)[None, :] + jnp.arange(2, dtype=jnp.int32)[:, None]
            ).reshape(2 * 16 * LAY_CHUNKS, 128)
    bidx = batch_p.reshape(POOL_ROWS, 128)

    ones8 = jnp.ones((128, 8), f32)
    zer8 = jnp.zeros((ROWS_PER_TILE, 8), f32)
    zer32 = jnp.zeros((ROWS_PER_TILE, 32), f32)
    zsum = jnp.zeros((GPAD, HID), f32)
    zcnt = jnp.zeros((GPAD, 8), f32)

    degp = _sc_deg(didx_deg, ones8, zer8)
    dinv, xw0, y0 = _tc_prep0(degp, x_p, W0)

    w_stack = jnp.stack([W1, W2, jnp.eye(HID, dtype=f32)])
    b_stack = jnp.stack([b0.reshape(1, HID), b1.reshape(1, HID),
                         b2.reshape(1, HID)])

    def step(i, carry):
        y, xw = carry
        scat = _sc_layer(y.reshape(2 * NPAD, 32), gidx, didx_lay, zer32)
        w_i = lax.dynamic_index_in_dim(w_stack, i, keepdims=False)
        b_i = lax.dynamic_index_in_dim(b_stack, i, keepdims=False)
        y_n, xw_n = _tc_mid(scat, xw, dinv, b_i, w_i)
        return (y_n, xw_n)

    _, h3 = lax.fori_loop(0, 3, step, (y0, xw0))

    sums, cnts = _sc_pool(h3, bidx, ones8, zsum, zcnt)
    cls, reg = _tc_head(sums, cnts, Wc1, bc1.reshape(1, HID // 2), Wc2,
                        bc2.reshape(1, 10), Wr1, br1.reshape(1, HID // 2),
                        Wr2, br2.reshape(1, 1))
    return cls[:G], reg[:G]


# speedup vs baseline: 12.8619x; 1.0600x over previous
import functools

import jax
import jax.numpy as jnp
from jax import lax
from jax.experimental import pallas as pl
from jax.experimental.pallas import tpu as pltpu
from jax.experimental.pallas import tpu_sc as plsc

N = 50000
E = 800000
G = 500
HID = 64

NPAD = 51200
EPAD = 819200
GPAD = 512
TRASH = NPAD - 1

ROWS_PER_TILE = NPAD // 16
DEG_CHUNKS = EPAD // (32 * 128)
LAY_CHUNKS = EPAD // (16 * 128)
POOL_ROWS = 416

_R = 512
_GRID = NPAD // _R

_SC_MESH = plsc.VectorSubcoreMesh(core_axis_name="c", subcore_axis_name="s")


@functools.partial(
    pl.kernel,
    out_type=jax.ShapeDtypeStruct((2 * NPAD, 8), jnp.float32),
    mesh=_SC_MESH,
    compiler_params=pltpu.CompilerParams(use_tc_tiling_on_sc=False),
    scratch_types=[
        pltpu.VMEM((DEG_CHUNKS, 128), jnp.int32),
        pltpu.VMEM((128, 8), jnp.float32),
        pltpu.VMEM_SHARED((NPAD, 8), jnp.float32),
        pltpu.SemaphoreType.DMA,
    ],
)
def _sc_deg(didx_hbm, ones_hbm, zeros_hbm, out_hbm, didx_v, ones_v, accum,
            dsem):
    c = lax.axis_index("c")
    s = lax.axis_index("s")
    w = s * 2 + c
    stripe = pl.multiple_of(s * ROWS_PER_TILE, 8)
    out_off = pl.multiple_of(c * NPAD + s * ROWS_PER_TILE, 8)
    pltpu.sync_copy(didx_hbm.at[pl.ds(pl.multiple_of(w * DEG_CHUNKS, 8), DEG_CHUNKS)],
                    didx_v)
    pltpu.sync_copy(ones_hbm, ones_v)
    pltpu.sync_copy(zeros_hbm, accum.at[pl.ds(stripe, ROWS_PER_TILE)])
    plsc.subcore_barrier()

    def body(g, carry):
        cps = [
            pltpu.async_copy(ones_v, accum.at[didx_v.at[g * 8 + k]], dsem,
                             add=True)
            for k in range(8)
        ]
        for cp in cps:
            cp.wait()
        return carry

    lax.fori_loop(0, DEG_CHUNKS // 8, body, 0)
    plsc.subcore_barrier()
    pltpu.sync_copy(
        accum.at[pl.ds(stripe, ROWS_PER_TILE)],
        out_hbm.at[pl.ds(out_off, ROWS_PER_TILE)],
    )


_KRING = 4
_IBLK = 40
_IBLK_N = LAY_CHUNKS // _IBLK
_RPB = _IBLK // _KRING


@functools.partial(
    pl.kernel,
    out_type=jax.ShapeDtypeStruct((2 * NPAD, 32), jnp.float32),
    mesh=_SC_MESH,
    compiler_params=pltpu.CompilerParams(use_tc_tiling_on_sc=False),
    scratch_types=[
        pltpu.VMEM((_IBLK, 128), jnp.int32),
        pltpu.VMEM((_IBLK, 128), jnp.int32),
        pltpu.VMEM((_KRING, 128, 32), jnp.float32),
        pltpu.VMEM_SHARED((NPAD, 32), jnp.float32),
        pltpu.SemaphoreType.DMA((_KRING,)),
        pltpu.SemaphoreType.DMA((_KRING,)),
    ],
)
def _sc_layer(yflat_hbm, gidx_hbm, didx_hbm, zeros_hbm, out_hbm,
              gidx_v, didx_v, rows_v, accum, gsem, ssem):
    c = lax.axis_index("c")
    s = lax.axis_index("s")
    stripe = pl.multiple_of(s * ROWS_PER_TILE, 8)
    out_off = pl.multiple_of(c * NPAD + s * ROWS_PER_TILE, 8)
    idx_base = pl.multiple_of((c * 16 + s) * LAY_CHUNKS, 8)
    didx_base = pl.multiple_of(s * LAY_CHUNKS, 8)
    pltpu.sync_copy(zeros_hbm, accum.at[pl.ds(stripe, ROWS_PER_TILE)])
    plsc.subcore_barrier()

    def drain_gather(b):
        pltpu.make_async_copy(yflat_hbm.at[gidx_v.at[0]], rows_v.at[b],
                              gsem.at[b]).wait()

    def drain_scatter(b):
        pltpu.make_async_copy(rows_v.at[b], accum.at[didx_v.at[0]],
                              ssem.at[b]).wait()

    def block_body(blk, carry):
        boff = pl.multiple_of(blk * _IBLK, 8)
        pltpu.sync_copy(gidx_hbm.at[pl.ds(idx_base + boff, _IBLK)], gidx_v)
        pltpu.sync_copy(didx_hbm.at[pl.ds(didx_base + boff, _IBLK)], didx_v)

        def round_body(r, carry2):
            for b in range(_KRING):
                @pl.when(r > 0)
                def _():
                    drain_scatter(b)
                pltpu.async_copy(yflat_hbm.at[gidx_v.at[r * _KRING + b]],
                                 rows_v.at[b], gsem.at[b])
            for b in range(_KRING):
                drain_gather(b)
                pltpu.async_copy(rows_v.at[b],
                                 accum.at[didx_v.at[r * _KRING + b]],
                                 ssem.at[b], add=True)
            return carry2

        lax.fori_loop(0, _RPB, round_body, 0)
        for b in range(_KRING):
            drain_scatter(b)
        return carry

    lax.fori_loop(0, _IBLK_N, block_body, 0)
    plsc.subcore_barrier()
    pltpu.sync_copy(
        accum.at[pl.ds(stripe, ROWS_PER_TILE)],
        out_hbm.at[pl.ds(out_off, ROWS_PER_TILE)],
    )


@functools.partial(
    pl.kernel,
    out_type=(
        jax.ShapeDtypeStruct((2 * GPAD, HID), jnp.float32),
        jax.ShapeDtypeStruct((2 * GPAD, 8), jnp.float32),
    ),
    mesh=_SC_MESH,
    compiler_params=pltpu.CompilerParams(use_tc_tiling_on_sc=False),
    scratch_types=[
        pltpu.VMEM((24, 128), jnp.int32),
        pltpu.VMEM((128, HID), jnp.float32),
        pltpu.VMEM((128, 8), jnp.float32),
        pltpu.VMEM_SHARED((GPAD, HID), jnp.float32),
        pltpu.VMEM_SHARED((GPAD, 8), jnp.float32),
    ],
)
def _sc_pool(h3_hbm, bidx_hbm, ones_hbm, zsum_hbm, zcnt_hbm,
             sums_hbm, cnts_hbm, bidx_v, rows_v, ones_v, accum, cacc):
    c = lax.axis_index("c")
    s = lax.axis_index("s")
    nch = jnp.where(s < 8, 13, 12)
    base = c * 200 + s * 12 + jnp.minimum(s, 8)
    base_al = pl.multiple_of((base // 8) * 8, 8)
    off = base - base_al
    pltpu.sync_copy(bidx_hbm.at[pl.ds(base_al, 24)], bidx_v)
    pltpu.sync_copy(ones_hbm, ones_v)

    @pl.when(s == 0)
    def _():
        pltpu.sync_copy(zsum_hbm, accum)
        pltpu.sync_copy(zcnt_hbm, cacc)

    plsc.subcore_barrier()

    def body(j, carry):
        ch = pl.multiple_of((base + j) * 128, 128)
        pltpu.sync_copy(h3_hbm.at[pl.ds(ch, 128)], rows_v)
        pltpu.sync_copy(rows_v, accum.at[bidx_v.at[off + j]], add=True)
        pltpu.sync_copy(ones_v, cacc.at[bidx_v.at[off + j]], add=True)
        return carry

    lax.fori_loop(0, nch, body, 0)
    plsc.subcore_barrier()
    pltpu.sync_copy(accum.at[pl.ds(pl.multiple_of(s * 32, 8), 32)],
                    sums_hbm.at[pl.ds(pl.multiple_of(c * GPAD + s * 32, 8), 32)])
    pltpu.sync_copy(cacc.at[pl.ds(pl.multiple_of(s * 32, 8), 32)],
                    cnts_hbm.at[pl.ds(pl.multiple_of(c * GPAD + s * 32, 8), 32)])


def _tc_prep0(degp, x, w0):
    def body(d0_ref, d1_ref, x_ref, w0_ref, dinv_ref, xw_ref, y_ref):
        deg = d0_ref[:, 0:1] + d1_ref[:, 0:1] + 1.0
        dinv = lax.rsqrt(deg)
        xw = jnp.dot(x_ref[...], w0_ref[...], preferred_element_type=jnp.float32)
        dinv_ref[...] = dinv
        xw_ref[...] = xw
        y_ref[...] = dinv * xw

    return pl.pallas_call(
        body,
        grid=(_GRID,),
        in_specs=[
            pl.BlockSpec((_R, 8), lambda i: (i, 0)),
            pl.BlockSpec((_R, 8), lambda i: (i + _GRID, 0)),
            pl.BlockSpec((_R, 3), lambda i: (i, 0)),
            pl.BlockSpec((3, HID), lambda i: (0, 0)),
        ],
        out_specs=[
            pl.BlockSpec((_R, 1), lambda i: (i, 0)),
            pl.BlockSpec((_R, HID), lambda i: (i, 0)),
            pl.BlockSpec((_R, HID), lambda i: (i, 0)),
        ],
        out_shape=[
            jax.ShapeDtypeStruct((NPAD, 1), jnp.float32),
            jax.ShapeDtypeStruct((NPAD, HID), jnp.float32),
            jax.ShapeDtypeStruct((NPAD, HID), jnp.float32),
        ],
    )(degp, degp, x, w0)


def _tc_mid(scat, xwp, dinv, b, w):
    def body(s0_ref, s1_ref, xwp_ref, dinv_ref, b_ref, w_ref, y_ref, xw_ref):
        dv = dinv_ref[...]
        sc = jnp.concatenate([s0_ref[...], s1_ref[...]], axis=1)
        h = jnp.maximum(dv * sc + dv * dv * xwp_ref[...] + b_ref[...], 0.0)
        xwn = jnp.dot(h, w_ref[...], preferred_element_type=jnp.float32)
        xw_ref[...] = xwn
        y_ref[...] = dv * xwn

    return pl.pallas_call(
        body,
        grid=(_GRID,),
        in_specs=[
            pl.BlockSpec((_R, 32), lambda i: (i, 0)),
            pl.BlockSpec((_R, 32), lambda i: (i + _GRID, 0)),
            pl.BlockSpec((_R, HID), lambda i: (i, 0)),
            pl.BlockSpec((_R, 1), lambda i: (i, 0)),
            pl.BlockSpec((1, HID), lambda i: (0, 0)),
            pl.BlockSpec((HID, HID), lambda i: (0, 0)),
        ],
        out_specs=[
            pl.BlockSpec((_R, HID), lambda i: (i, 0)),
            pl.BlockSpec((_R, HID), lambda i: (i, 0)),
        ],
        out_shape=[
            jax.ShapeDtypeStruct((NPAD, HID), jnp.float32),
            jax.ShapeDtypeStruct((NPAD, HID), jnp.float32),
        ],
    )(scat, scat, xwp, dinv, b, w)


def _tc_head(sums, cnts, wc1, bc1, wc2, bc2, wr1, br1, wr2, br2):
    def body(p_ref, c_ref, wc1_ref, bc1_ref, wc2_ref, bc2_ref,
             wr1_ref, br1_ref, wr2_ref, br2_ref, cls_ref, reg_ref):
        ssum = p_ref[0:GPAD, :] + p_ref[GPAD:2 * GPAD, :]
        cnt = c_ref[0:GPAD, 0:1] + c_ref[GPAD:2 * GPAD, 0:1]
        pooled = ssum / jnp.maximum(cnt, 1.0)
        hc = jnp.maximum(
            jnp.dot(pooled, wc1_ref[...], preferred_element_type=jnp.float32)
            + bc1_ref[...], 0.0)
        cls_ref[...] = (jnp.dot(hc, wc2_ref[...], preferred_element_type=jnp.float32)
                        + bc2_ref[...])
        hr = jnp.maximum(
            jnp.dot(pooled, wr1_ref[...], preferred_element_type=jnp.float32)
            + br1_ref[...], 0.0)
        reg_ref[...] = (jnp.dot(hr, wr2_ref[...], preferred_element_type=jnp.float32)
                        + br2_ref[...])

    return pl.pallas_call(
        body,
        out_shape=[
            jax.ShapeDtypeStruct((GPAD, 10), jnp.float32),
            jax.ShapeDtypeStruct((GPAD, 1), jnp.float32),
        ],
    )(sums, cnts, wc1, bc1, wc2, bc2, wr1, br1, wr2, br2)


def kernel(x, W0, b0, W1, b1, W2, b2, Wc1, bc1, Wc2, bc2, Wr1, br1, Wr2, br2,
           edge_index, batch):
    f32 = jnp.float32
    src = edge_index[0]
    dst = edge_index[1]
    pad_e = jnp.full((EPAD - E,), TRASH, jnp.int32)
    src_p = jnp.concatenate([src, pad_e])
    dst_p = jnp.concatenate([dst, pad_e])
    x_p = jnp.pad(x, ((0, NPAD - N), (0, 0)))
    batch_p = jnp.concatenate(
        [batch, jnp.full((POOL_ROWS * 128 - N,), GPAD - 1, jnp.int32)])

    didx_deg = dst_p.reshape(32 * DEG_CHUNKS, 128)
    didx_lay = dst_p.reshape(16 * LAY_CHUNKS, 128)
    gidx = ((2 * src_p)[None, :] + jnp.arange(2, dtype=jnp.int32)[:, None]
            ).reshape(2 * 16 * LAY_CHUNKS, 128)
    bidx = batch_p.reshape(POOL_ROWS, 128)

    ones8 = jnp.ones((128, 8), f32)
    zer8 = jnp.zeros((ROWS_PER_TILE, 8), f32)
    zer32 = jnp.zeros((ROWS_PER_TILE, 32), f32)
    zsum = jnp.zeros((GPAD, HID), f32)
    zcnt = jnp.zeros((GPAD, 8), f32)

    degp = _sc_deg(didx_deg, ones8, zer8)
    dinv, xw0, y0 = _tc_prep0(degp, x_p, W0)

    w_stack = jnp.stack([W1, W2, jnp.eye(HID, dtype=f32)])
    b_stack = jnp.stack([b0.reshape(1, HID), b1.reshape(1, HID),
                         b2.reshape(1, HID)])

    def step(i, carry):
        y, xw = carry
        scat = _sc_layer(y.reshape(2 * NPAD, 32), gidx, didx_lay, zer32)
        w_i = lax.dynamic_index_in_dim(w_stack, i, keepdims=False)
        b_i = lax.dynamic_index_in_dim(b_stack, i, keepdims=False)
        y_n, xw_n = _tc_mid(scat, xw, dinv, b_i, w_i)
        return (y_n, xw_n)

    _, h3 = lax.fori_loop(0, 3, step, (y0, xw0))

    sums, cnts = _sc_pool(h3, bidx, ones8, zsum, zcnt)
    cls, reg = _tc_head(sums, cnts, Wc1, bc1.reshape(1, HID // 2), Wc2,
                        bc2.reshape(1, 10), Wr1, br1.reshape(1, HID // 2),
                        Wr2, br2.reshape(1, 1))
    return cls[:G], reg[:G]


# speedup vs baseline: 15.3414x; 1.1928x over previous
import functools

import jax
import jax.numpy as jnp
from jax import lax
from jax.experimental import pallas as pl
from jax.experimental.pallas import tpu as pltpu
from jax.experimental.pallas import tpu_sc as plsc

N = 50000
E = 800000
G = 500
HID = 64

NPAD = 51200
EPAD = 819200
GPAD = 512
TRASH = NPAD - 1

ROWS_PER_TILE = NPAD // 16
DEG_CHUNKS = EPAD // (32 * 128)
LAY_CHUNKS = EPAD // (16 * 128)
POOL_ROWS = 416

_R = 1024
_GRID = NPAD // _R

_SC_MESH = plsc.VectorSubcoreMesh(core_axis_name="c", subcore_axis_name="s")


@functools.partial(
    pl.kernel,
    out_type=jax.ShapeDtypeStruct((2 * NPAD, 8), jnp.float32),
    mesh=_SC_MESH,
    compiler_params=pltpu.CompilerParams(use_tc_tiling_on_sc=False),
    scratch_types=[
        pltpu.VMEM((DEG_CHUNKS, 128), jnp.int32),
        pltpu.VMEM((128, 8), jnp.float32),
        pltpu.VMEM_SHARED((NPAD, 8), jnp.float32),
        pltpu.SemaphoreType.DMA,
    ],
)
def _sc_deg(didx_hbm, ones_hbm, zeros_hbm, out_hbm, didx_v, ones_v, accum,
            dsem):
    c = lax.axis_index("c")
    s = lax.axis_index("s")
    w = s * 2 + c
    stripe = pl.multiple_of(s * ROWS_PER_TILE, 8)
    out_off = pl.multiple_of(c * NPAD + s * ROWS_PER_TILE, 8)
    pltpu.sync_copy(didx_hbm.at[pl.ds(pl.multiple_of(w * DEG_CHUNKS, 8), DEG_CHUNKS)],
                    didx_v)
    pltpu.sync_copy(ones_hbm, ones_v)
    pltpu.sync_copy(zeros_hbm, accum.at[pl.ds(stripe, ROWS_PER_TILE)])
    plsc.subcore_barrier()

    def body(g, carry):
        cps = [
            pltpu.async_copy(ones_v, accum.at[didx_v.at[g * 8 + k]], dsem,
                             add=True)
            for k in range(8)
        ]
        for cp in cps:
            cp.wait()
        return carry

    lax.fori_loop(0, DEG_CHUNKS // 8, body, 0)
    plsc.subcore_barrier()
    pltpu.sync_copy(
        accum.at[pl.ds(stripe, ROWS_PER_TILE)],
        out_hbm.at[pl.ds(out_off, ROWS_PER_TILE)],
    )


_KRING = 4
_IBLK = 40
_IBLK_N = LAY_CHUNKS // _IBLK
_RPB = _IBLK // _KRING


@functools.partial(
    pl.kernel,
    out_type=jax.ShapeDtypeStruct((2 * NPAD, 32), jnp.float32),
    mesh=_SC_MESH,
    compiler_params=pltpu.CompilerParams(use_tc_tiling_on_sc=False),
    scratch_types=[
        pltpu.VMEM((_IBLK, 128), jnp.int32),
        pltpu.VMEM((_IBLK, 128), jnp.int32),
        pltpu.VMEM((_KRING, 128, 32), jnp.float32),
        pltpu.VMEM_SHARED((NPAD, 32), jnp.float32),
        pltpu.SemaphoreType.DMA((_KRING,)),
        pltpu.SemaphoreType.DMA((_KRING,)),
    ],
)
def _sc_layer(yflat_hbm, gidx_hbm, didx_hbm, zeros_hbm, out_hbm,
              gidx_v, didx_v, rows_v, accum, gsem, ssem):
    c = lax.axis_index("c")
    s = lax.axis_index("s")
    stripe = pl.multiple_of(s * ROWS_PER_TILE, 8)
    out_off = pl.multiple_of(c * NPAD + s * ROWS_PER_TILE, 8)
    idx_base = pl.multiple_of((c * 16 + s) * LAY_CHUNKS, 8)
    didx_base = pl.multiple_of(s * LAY_CHUNKS, 8)
    pltpu.sync_copy(zeros_hbm, accum.at[pl.ds(stripe, ROWS_PER_TILE)])
    plsc.subcore_barrier()

    def drain_gather(b):
        pltpu.make_async_copy(yflat_hbm.at[gidx_v.at[0]], rows_v.at[b],
                              gsem.at[b]).wait()

    def drain_scatter(b):
        pltpu.make_async_copy(rows_v.at[b], accum.at[didx_v.at[0]],
                              ssem.at[b]).wait()

    def block_body(blk, carry):
        boff = pl.multiple_of(blk * _IBLK, 8)
        pltpu.sync_copy(gidx_hbm.at[pl.ds(idx_base + boff, _IBLK)], gidx_v)
        pltpu.sync_copy(didx_hbm.at[pl.ds(didx_base + boff, _IBLK)], didx_v)

        def round_body(r, carry2):
            for b in range(_KRING):
                @pl.when(r > 0)
                def _():
                    drain_scatter(b)
                pltpu.async_copy(yflat_hbm.at[gidx_v.at[r * _KRING + b]],
                                 rows_v.at[b], gsem.at[b])
            for b in range(_KRING):
                drain_gather(b)
                pltpu.async_copy(rows_v.at[b],
                                 accum.at[didx_v.at[r * _KRING + b]],
                                 ssem.at[b], add=True)
            return carry2

        lax.fori_loop(0, _RPB, round_body, 0)
        for b in range(_KRING):
            drain_scatter(b)
        return carry

    lax.fori_loop(0, _IBLK_N, block_body, 0)
    plsc.subcore_barrier()
    pltpu.sync_copy(
        accum.at[pl.ds(stripe, ROWS_PER_TILE)],
        out_hbm.at[pl.ds(out_off, ROWS_PER_TILE)],
    )


@functools.partial(
    pl.kernel,
    out_type=(
        jax.ShapeDtypeStruct((2 * GPAD, HID), jnp.float32),
        jax.ShapeDtypeStruct((2 * GPAD, 8), jnp.float32),
    ),
    mesh=_SC_MESH,
    compiler_params=pltpu.CompilerParams(use_tc_tiling_on_sc=False),
    scratch_types=[
        pltpu.VMEM((24, 128), jnp.int32),
        pltpu.VMEM((128, HID), jnp.float32),
        pltpu.VMEM((128, 8), jnp.float32),
        pltpu.VMEM_SHARED((GPAD, HID), jnp.float32),
        pltpu.VMEM_SHARED((GPAD, 8), jnp.float32),
    ],
)
def _sc_pool(h3_hbm, bidx_hbm, ones_hbm, zsum_hbm, zcnt_hbm,
             sums_hbm, cnts_hbm, bidx_v, rows_v, ones_v, accum, cacc):
    c = lax.axis_index("c")
    s = lax.axis_index("s")
    nch = jnp.where(s < 8, 13, 12)
    base = c * 200 + s * 12 + jnp.minimum(s, 8)
    base_al = pl.multiple_of((base // 8) * 8, 8)
    off = base - base_al
    pltpu.sync_copy(bidx_hbm.at[pl.ds(base_al, 24)], bidx_v)
    pltpu.sync_copy(ones_hbm, ones_v)

    @pl.when(s == 0)
    def _():
        pltpu.sync_copy(zsum_hbm, accum)
        pltpu.sync_copy(zcnt_hbm, cacc)

    plsc.subcore_barrier()

    def body(j, carry):
        ch = pl.multiple_of((base + j) * 128, 128)
        pltpu.sync_copy(h3_hbm.at[pl.ds(ch, 128)], rows_v)
        pltpu.sync_copy(rows_v, accum.at[bidx_v.at[off + j]], add=True)
        pltpu.sync_copy(ones_v, cacc.at[bidx_v.at[off + j]], add=True)
        return carry

    lax.fori_loop(0, nch, body, 0)
    plsc.subcore_barrier()
    pltpu.sync_copy(accum.at[pl.ds(pl.multiple_of(s * 32, 8), 32)],
                    sums_hbm.at[pl.ds(pl.multiple_of(c * GPAD + s * 32, 8), 32)])
    pltpu.sync_copy(cacc.at[pl.ds(pl.multiple_of(s * 32, 8), 32)],
                    cnts_hbm.at[pl.ds(pl.multiple_of(c * GPAD + s * 32, 8), 32)])


def _tc_prep0(degp, x, w0):
    def body(d0_ref, d1_ref, x_ref, w0_ref, dinv_ref, xw_ref, y_ref):
        deg = d0_ref[:, 0:1] + d1_ref[:, 0:1] + 1.0
        dinv = lax.rsqrt(deg)
        xw = jnp.dot(x_ref[...], w0_ref[...], preferred_element_type=jnp.float32)
        dinv_ref[...] = jnp.broadcast_to(dinv, (_R, 8))
        xw_ref[...] = xw
        yv = dinv * xw
        y_ref[...] = jnp.concatenate([yv[None, :, :32], yv[None, :, 32:]],
                                     axis=0)

    return pl.pallas_call(
        body,
        grid=(_GRID,),
        in_specs=[
            pl.BlockSpec((_R, 8), lambda i: (i, 0)),
            pl.BlockSpec((_R, 8), lambda i: (i + _GRID, 0)),
            pl.BlockSpec((_R, 3), lambda i: (i, 0)),
            pl.BlockSpec((3, HID), lambda i: (0, 0)),
        ],
        out_specs=[
            pl.BlockSpec((_R, 8), lambda i: (i, 0)),
            pl.BlockSpec((_R, HID), lambda i: (i, 0)),
            pl.BlockSpec((2, _R, 32), lambda i: (0, i, 0)),
        ],
        out_shape=[
            jax.ShapeDtypeStruct((NPAD, 8), jnp.float32),
            jax.ShapeDtypeStruct((NPAD, HID), jnp.float32),
            jax.ShapeDtypeStruct((2, NPAD, 32), jnp.float32),
        ],
    )(degp, degp, x, w0)


def _tc_mid(scat, xwp, dinv, b, w):
    def body(s0_ref, s1_ref, xwp_ref, dinv_ref, b_ref, w_ref, y_ref, xw_ref):
        dv = dinv_ref[:, 0:1]
        sc = jnp.concatenate([s0_ref[...], s1_ref[...]], axis=1)
        h = jnp.maximum(dv * sc + dv * dv * xwp_ref[...] + b_ref[...], 0.0)
        xwn = jnp.dot(h, w_ref[...], preferred_element_type=jnp.float32)
        xw_ref[...] = xwn
        yv = dv * xwn
        y_ref[...] = jnp.concatenate([yv[None, :, :32], yv[None, :, 32:]],
                                     axis=0)

    return pl.pallas_call(
        body,
        grid=(_GRID,),
        in_specs=[
            pl.BlockSpec((_R, 32), lambda i: (i, 0)),
            pl.BlockSpec((_R, 32), lambda i: (i + _GRID, 0)),
            pl.BlockSpec((_R, HID), lambda i: (i, 0)),
            pl.BlockSpec((_R, 8), lambda i: (i, 0)),
            pl.BlockSpec((1, HID), lambda i: (0, 0)),
            pl.BlockSpec((HID, HID), lambda i: (0, 0)),
        ],
        out_specs=[
            pl.BlockSpec((2, _R, 32), lambda i: (0, i, 0)),
            pl.BlockSpec((_R, HID), lambda i: (i, 0)),
        ],
        out_shape=[
            jax.ShapeDtypeStruct((2, NPAD, 32), jnp.float32),
            jax.ShapeDtypeStruct((NPAD, HID), jnp.float32),
        ],
    )(scat, scat, xwp, dinv, b, w)


def _tc_last(scat, xwp, dinv, b):
    def body(s0_ref, s1_ref, xwp_ref, dinv_ref, b_ref, h_ref):
        dv = dinv_ref[:, 0:1]
        sc = jnp.concatenate([s0_ref[...], s1_ref[...]], axis=1)
        h_ref[...] = jnp.maximum(dv * sc + dv * dv * xwp_ref[...] + b_ref[...],
                                 0.0)

    return pl.pallas_call(
        body,
        grid=(_GRID,),
        in_specs=[
            pl.BlockSpec((_R, 32), lambda i: (i, 0)),
            pl.BlockSpec((_R, 32), lambda i: (i + _GRID, 0)),
            pl.BlockSpec((_R, HID), lambda i: (i, 0)),
            pl.BlockSpec((_R, 8), lambda i: (i, 0)),
            pl.BlockSpec((1, HID), lambda i: (0, 0)),
        ],
        out_specs=pl.BlockSpec((_R, HID), lambda i: (i, 0)),
        out_shape=jax.ShapeDtypeStruct((NPAD, HID), jnp.float32),
    )(scat, scat, xwp, dinv, b)


def _tc_head(sums, cnts, wc1, bc1, wc2, bc2, wr1, br1, wr2, br2):
    def body(p_ref, c_ref, wc1_ref, bc1_ref, wc2_ref, bc2_ref,
             wr1_ref, br1_ref, wr2_ref, br2_ref, cls_ref, reg_ref):
        ssum = p_ref[0:GPAD, :] + p_ref[GPAD:2 * GPAD, :]
        cnt = c_ref[0:GPAD, 0:1] + c_ref[GPAD:2 * GPAD, 0:1]
        pooled = ssum / jnp.maximum(cnt, 1.0)
        hc = jnp.maximum(
            jnp.dot(pooled, wc1_ref[...], preferred_element_type=jnp.float32)
            + bc1_ref[...], 0.0)
        cls_ref[...] = (jnp.dot(hc, wc2_ref[...], preferred_element_type=jnp.float32)
                        + bc2_ref[...])
        hr = jnp.maximum(
            jnp.dot(pooled, wr1_ref[...], preferred_element_type=jnp.float32)
            + br1_ref[...], 0.0)
        reg_ref[...] = (jnp.dot(hr, wr2_ref[...], preferred_element_type=jnp.float32)
                        + br2_ref[...])

    return pl.pallas_call(
        body,
        out_shape=[
            jax.ShapeDtypeStruct((GPAD, 10), jnp.float32),
            jax.ShapeDtypeStruct((GPAD, 1), jnp.float32),
        ],
    )(sums, cnts, wc1, bc1, wc2, bc2, wr1, br1, wr2, br2)


def kernel(x, W0, b0, W1, b1, W2, b2, Wc1, bc1, Wc2, bc2, Wr1, br1, Wr2, br2,
           edge_index, batch):
    f32 = jnp.float32
    src = edge_index[0]
    dst = edge_index[1]
    pad_e = jnp.full((EPAD - E,), TRASH, jnp.int32)
    src_p = jnp.concatenate([src, pad_e])
    dst_p = jnp.concatenate([dst, pad_e])
    x_p = jnp.pad(x, ((0, NPAD - N), (0, 0)))
    batch_p = jnp.concatenate(
        [batch, jnp.full((POOL_ROWS * 128 - N,), GPAD - 1, jnp.int32)])

    didx_deg = dst_p.reshape(32 * DEG_CHUNKS, 128)
    didx_lay = dst_p.reshape(16 * LAY_CHUNKS, 128)
    gidx = (src_p[None, :] + jnp.array([0, NPAD], jnp.int32)[:, None]
            ).reshape(2 * 16 * LAY_CHUNKS, 128)
    bidx = batch_p.reshape(POOL_ROWS, 128)

    ones8 = jnp.ones((128, 8), f32)
    zer8 = jnp.zeros((ROWS_PER_TILE, 8), f32)
    zer32 = jnp.zeros((ROWS_PER_TILE, 32), f32)
    zsum = jnp.zeros((GPAD, HID), f32)
    zcnt = jnp.zeros((GPAD, 8), f32)

    degp = _sc_deg(didx_deg, ones8, zer8)
    dinv, xw0, y0 = _tc_prep0(degp, x_p, W0)

    scat0 = _sc_layer(y0.reshape(2 * NPAD, 32), gidx, didx_lay, zer32)
    y1, xw1 = _tc_mid(scat0, xw0, dinv, b0.reshape(1, HID), W1)
    scat1 = _sc_layer(y1.reshape(2 * NPAD, 32), gidx, didx_lay, zer32)
    y2, xw2 = _tc_mid(scat1, xw1, dinv, b1.reshape(1, HID), W2)
    scat2 = _sc_layer(y2.reshape(2 * NPAD, 32), gidx, didx_lay, zer32)
    h3 = _tc_last(scat2, xw2, dinv, b2.reshape(1, HID))

    sums, cnts = _sc_pool(h3, bidx, ones8, zsum, zcnt)
    cls, reg = _tc_head(sums, cnts, Wc1, bc1.reshape(1, HID // 2), Wc2,
                        bc2.reshape(1, 10), Wr1, br1.reshape(1, HID // 2),
                        Wr2, br2.reshape(1, 1))
    return cls[:G], reg[:G]


# speedup vs baseline: 15.7596x; 1.0273x over previous
import functools

import jax
import jax.numpy as jnp
from jax import lax
from jax.experimental import pallas as pl
from jax.experimental.pallas import tpu as pltpu
from jax.experimental.pallas import tpu_sc as plsc

N = 50000
E = 800000
G = 500
HID = 64

NPAD = 51200
EPAD = 819200
GPAD = 512
TRASH = NPAD - 1

ROWS_PER_TILE = NPAD // 16
DEG_CHUNKS = EPAD // (32 * 128)
LAY_CHUNKS = EPAD // (16 * 128)
POOL_ROWS = 416

_R = 2048
_GRID = NPAD // _R

_SC_MESH = plsc.VectorSubcoreMesh(core_axis_name="c", subcore_axis_name="s")


@functools.partial(
    pl.kernel,
    out_type=jax.ShapeDtypeStruct((2 * NPAD, 8), jnp.float32),
    mesh=_SC_MESH,
    compiler_params=pltpu.CompilerParams(use_tc_tiling_on_sc=False),
    scratch_types=[
        pltpu.VMEM((DEG_CHUNKS, 128), jnp.int32),
        pltpu.VMEM((128, 8), jnp.float32),
        pltpu.VMEM_SHARED((NPAD, 8), jnp.float32),
        pltpu.SemaphoreType.DMA,
    ],
)
def _sc_deg(didx_hbm, ones_hbm, zeros_hbm, out_hbm, didx_v, ones_v, accum,
            dsem):
    c = lax.axis_index("c")
    s = lax.axis_index("s")
    w = s * 2 + c
    stripe = pl.multiple_of(s * ROWS_PER_TILE, 8)
    out_off = pl.multiple_of(c * NPAD + s * ROWS_PER_TILE, 8)
    pltpu.sync_copy(didx_hbm.at[pl.ds(pl.multiple_of(w * DEG_CHUNKS, 8), DEG_CHUNKS)],
                    didx_v)
    pltpu.sync_copy(ones_hbm, ones_v)
    pltpu.sync_copy(zeros_hbm, accum.at[pl.ds(stripe, ROWS_PER_TILE)])
    plsc.subcore_barrier()

    def body(g, carry):
        cps = [
            pltpu.async_copy(ones_v, accum.at[didx_v.at[g * 8 + k]], dsem,
                             add=True)
            for k in range(8)
        ]
        for cp in cps:
            cp.wait()
        return carry

    lax.fori_loop(0, DEG_CHUNKS // 8, body, 0)
    plsc.subcore_barrier()
    pltpu.sync_copy(
        accum.at[pl.ds(stripe, ROWS_PER_TILE)],
        out_hbm.at[pl.ds(out_off, ROWS_PER_TILE)],
    )


_KRING = 4
_IBLK = 40
_IBLK_N = LAY_CHUNKS // _IBLK
_RPB = _IBLK // _KRING


@functools.partial(
    pl.kernel,
    out_type=jax.ShapeDtypeStruct((2 * NPAD, 32), jnp.float32),
    mesh=_SC_MESH,
    compiler_params=pltpu.CompilerParams(use_tc_tiling_on_sc=False),
    scratch_types=[
        pltpu.VMEM((_IBLK, 128), jnp.int32),
        pltpu.VMEM((_IBLK, 128), jnp.int32),
        pltpu.VMEM((_KRING, 128, 32), jnp.float32),
        pltpu.VMEM_SHARED((NPAD, 32), jnp.float32),
        pltpu.SemaphoreType.DMA((_KRING,)),
        pltpu.SemaphoreType.DMA((_KRING,)),
    ],
)
def _sc_layer(yflat_hbm, gidx_hbm, didx_hbm, zeros_hbm, out_hbm,
              gidx_v, didx_v, rows_v, accum, gsem, ssem):
    c = lax.axis_index("c")
    s = lax.axis_index("s")
    stripe = pl.multiple_of(s * ROWS_PER_TILE, 8)
    out_off = pl.multiple_of(c * NPAD + s * ROWS_PER_TILE, 8)
    idx_base = pl.multiple_of((c * 16 + s) * LAY_CHUNKS, 8)
    didx_base = pl.multiple_of(s * LAY_CHUNKS, 8)
    pltpu.sync_copy(zeros_hbm, accum.at[pl.ds(stripe, ROWS_PER_TILE)])
    plsc.subcore_barrier()

    def drain_gather(b):
        pltpu.make_async_copy(yflat_hbm.at[gidx_v.at[0]], rows_v.at[b],
                              gsem.at[b]).wait()

    def drain_scatter(b):
        pltpu.make_async_copy(rows_v.at[b], accum.at[didx_v.at[0]],
                              ssem.at[b]).wait()

    def block_body(blk, carry):
        boff = pl.multiple_of(blk * _IBLK, 8)
        pltpu.sync_copy(gidx_hbm.at[pl.ds(idx_base + boff, _IBLK)], gidx_v)
        pltpu.sync_copy(didx_hbm.at[pl.ds(didx_base + boff, _IBLK)], didx_v)

        def round_body(r, carry2):
            for b in range(_KRING):
                @pl.when(r > 0)
                def _():
                    drain_scatter(b)
                pltpu.async_copy(yflat_hbm.at[gidx_v.at[r * _KRING + b]],
                                 rows_v.at[b], gsem.at[b])
            for b in range(_KRING):
                drain_gather(b)
                pltpu.async_copy(rows_v.at[b],
                                 accum.at[didx_v.at[r * _KRING + b]],
                                 ssem.at[b], add=True)
            return carry2

        lax.fori_loop(0, _RPB, round_body, 0)
        for b in range(_KRING):
            drain_scatter(b)
        return carry

    lax.fori_loop(0, _IBLK_N, block_body, 0)
    plsc.subcore_barrier()
    pltpu.sync_copy(
        accum.at[pl.ds(stripe, ROWS_PER_TILE)],
        out_hbm.at[pl.ds(out_off, ROWS_PER_TILE)],
    )


@functools.partial(
    pl.kernel,
    out_type=(
        jax.ShapeDtypeStruct((2 * GPAD, HID), jnp.float32),
        jax.ShapeDtypeStruct((2 * GPAD, 8), jnp.float32),
    ),
    mesh=_SC_MESH,
    compiler_params=pltpu.CompilerParams(use_tc_tiling_on_sc=False),
    scratch_types=[
        pltpu.VMEM((24, 128), jnp.int32),
        pltpu.VMEM((128, HID), jnp.float32),
        pltpu.VMEM((128, 8), jnp.float32),
        pltpu.VMEM_SHARED((GPAD, HID), jnp.float32),
        pltpu.VMEM_SHARED((GPAD, 8), jnp.float32),
    ],
)
def _sc_pool(h3_hbm, bidx_hbm, ones_hbm, zsum_hbm, zcnt_hbm,
             sums_hbm, cnts_hbm, bidx_v, rows_v, ones_v, accum, cacc):
    c = lax.axis_index("c")
    s = lax.axis_index("s")
    nch = jnp.where(s < 8, 13, 12)
    base = c * 200 + s * 12 + jnp.minimum(s, 8)
    base_al = pl.multiple_of((base // 8) * 8, 8)
    off = base - base_al
    pltpu.sync_copy(bidx_hbm.at[pl.ds(base_al, 24)], bidx_v)
    pltpu.sync_copy(ones_hbm, ones_v)

    @pl.when(s == 0)
    def _():
        pltpu.sync_copy(zsum_hbm, accum)
        pltpu.sync_copy(zcnt_hbm, cacc)

    plsc.subcore_barrier()

    def body(j, carry):
        ch = pl.multiple_of((base + j) * 128, 128)
        pltpu.sync_copy(h3_hbm.at[pl.ds(ch, 128)], rows_v)
        pltpu.sync_copy(rows_v, accum.at[bidx_v.at[off + j]], add=True)
        pltpu.sync_copy(ones_v, cacc.at[bidx_v.at[off + j]], add=True)
        return carry

    lax.fori_loop(0, nch, body, 0)
    plsc.subcore_barrier()
    pltpu.sync_copy(accum.at[pl.ds(pl.multiple_of(s * 32, 8), 32)],
                    sums_hbm.at[pl.ds(pl.multiple_of(c * GPAD + s * 32, 8), 32)])
    pltpu.sync_copy(cacc.at[pl.ds(pl.multiple_of(s * 32, 8), 32)],
                    cnts_hbm.at[pl.ds(pl.multiple_of(c * GPAD + s * 32, 8), 32)])


def _tc_prep0(degp, x, w0):
    def body(d0_ref, d1_ref, x_ref, w0_ref, dinv_ref, xw_ref, y_ref):
        deg = d0_ref[:, 0:1] + d1_ref[:, 0:1] + 1.0
        dinv = lax.rsqrt(deg)
        xw = jnp.dot(x_ref[...], w0_ref[...], preferred_element_type=jnp.float32)
        dinv_ref[...] = jnp.broadcast_to(dinv, (_R, 8))
        xw_ref[...] = xw
        yv = dinv * xw
        y_ref[...] = jnp.concatenate([yv[None, :, :32], yv[None, :, 32:]],
                                     axis=0)

    return pl.pallas_call(
        body,
        grid=(_GRID,),
        in_specs=[
            pl.BlockSpec((_R, 8), lambda i: (i, 0)),
            pl.BlockSpec((_R, 8), lambda i: (i + _GRID, 0)),
            pl.BlockSpec((_R, 3), lambda i: (i, 0)),
            pl.BlockSpec((3, HID), lambda i: (0, 0)),
        ],
        out_specs=[
            pl.BlockSpec((_R, 8), lambda i: (i, 0)),
            pl.BlockSpec((_R, HID), lambda i: (i, 0)),
            pl.BlockSpec((2, _R, 32), lambda i: (0, i, 0)),
        ],
        out_shape=[
            jax.ShapeDtypeStruct((NPAD, 8), jnp.float32),
            jax.ShapeDtypeStruct((NPAD, HID), jnp.float32),
            jax.ShapeDtypeStruct((2, NPAD, 32), jnp.float32),
        ],
    )(degp, degp, x, w0)


def _tc_mid(scat, xwp, dinv, b, w):
    def body(s0_ref, s1_ref, xwp_ref, dinv_ref, b_ref, w_ref, y_ref, xw_ref):
        dv = dinv_ref[:, 0:1]
        sc = jnp.concatenate([s0_ref[...], s1_ref[...]], axis=1)
        h = jnp.maximum(dv * sc + dv * dv * xwp_ref[...] + b_ref[...], 0.0)
        xwn = jnp.dot(h, w_ref[...], preferred_element_type=jnp.float32)
        xw_ref[...] = xwn
        yv = dv * xwn
        y_ref[...] = jnp.concatenate([yv[None, :, :32], yv[None, :, 32:]],
                                     axis=0)

    return pl.pallas_call(
        body,
        grid=(_GRID,),
        in_specs=[
            pl.BlockSpec((_R, 32), lambda i: (i, 0)),
            pl.BlockSpec((_R, 32), lambda i: (i + _GRID, 0)),
            pl.BlockSpec((_R, HID), lambda i: (i, 0)),
            pl.BlockSpec((_R, 8), lambda i: (i, 0)),
            pl.BlockSpec((1, HID), lambda i: (0, 0)),
            pl.BlockSpec((HID, HID), lambda i: (0, 0)),
        ],
        out_specs=[
            pl.BlockSpec((2, _R, 32), lambda i: (0, i, 0)),
            pl.BlockSpec((_R, HID), lambda i: (i, 0)),
        ],
        out_shape=[
            jax.ShapeDtypeStruct((2, NPAD, 32), jnp.float32),
            jax.ShapeDtypeStruct((NPAD, HID), jnp.float32),
        ],
    )(scat, scat, xwp, dinv, b, w)


def _tc_last(scat, xwp, dinv, b):
    def body(s0_ref, s1_ref, xwp_ref, dinv_ref, b_ref, h_ref):
        dv = dinv_ref[:, 0:1]
        sc = jnp.concatenate([s0_ref[...], s1_ref[...]], axis=1)
        h_ref[...] = jnp.maximum(dv * sc + dv * dv * xwp_ref[...] + b_ref[...],
                                 0.0)

    return pl.pallas_call(
        body,
        grid=(_GRID,),
        in_specs=[
            pl.BlockSpec((_R, 32), lambda i: (i, 0)),
            pl.BlockSpec((_R, 32), lambda i: (i + _GRID, 0)),
            pl.BlockSpec((_R, HID), lambda i: (i, 0)),
            pl.BlockSpec((_R, 8), lambda i: (i, 0)),
            pl.BlockSpec((1, HID), lambda i: (0, 0)),
        ],
        out_specs=pl.BlockSpec((_R, HID), lambda i: (i, 0)),
        out_shape=jax.ShapeDtypeStruct((NPAD, HID), jnp.float32),
    )(scat, scat, xwp, dinv, b)


def _tc_head(sums, cnts, wc1, bc1, wc2, bc2, wr1, br1, wr2, br2):
    def body(p_ref, c_ref, wc1_ref, bc1_ref, wc2_ref, bc2_ref,
             wr1_ref, br1_ref, wr2_ref, br2_ref, cls_ref, reg_ref):
        ssum = p_ref[0:GPAD, :] + p_ref[GPAD:2 * GPAD, :]
        cnt = c_ref[0:GPAD, 0:1] + c_ref[GPAD:2 * GPAD, 0:1]
        pooled = ssum / jnp.maximum(cnt, 1.0)
        hc = jnp.maximum(
            jnp.dot(pooled, wc1_ref[...], preferred_element_type=jnp.float32)
            + bc1_ref[...], 0.0)
        cls_ref[...] = (jnp.dot(hc, wc2_ref[...], preferred_element_type=jnp.float32)
                        + bc2_ref[...])
        hr = jnp.maximum(
            jnp.dot(pooled, wr1_ref[...], preferred_element_type=jnp.float32)
            + br1_ref[...], 0.0)
        reg_ref[...] = (jnp.dot(hr, wr2_ref[...], preferred_element_type=jnp.float32)
                        + br2_ref[...])

    return pl.pallas_call(
        body,
        out_shape=[
            jax.ShapeDtypeStruct((GPAD, 10), jnp.float32),
            jax.ShapeDtypeStruct((GPAD, 1), jnp.float32),
        ],
    )(sums, cnts, wc1, bc1, wc2, bc2, wr1, br1, wr2, br2)


def kernel(x, W0, b0, W1, b1, W2, b2, Wc1, bc1, Wc2, bc2, Wr1, br1, Wr2, br2,
           edge_index, batch):
    f32 = jnp.float32
    src = edge_index[0]
    dst = edge_index[1]
    pad_e = jnp.full((EPAD - E,), TRASH, jnp.int32)
    src_p = jnp.concatenate([src, pad_e])
    dst_p = jnp.concatenate([dst, pad_e])
    x_p = jnp.pad(x, ((0, NPAD - N), (0, 0)))
    batch_p = jnp.concatenate(
        [batch, jnp.full((POOL_ROWS * 128 - N,), GPAD - 1, jnp.int32)])

    didx_deg = dst_p.reshape(32 * DEG_CHUNKS, 128)
    didx_lay = dst_p.reshape(16 * LAY_CHUNKS, 128)
    gidx = (src_p[None, :] + jnp.array([0, NPAD], jnp.int32)[:, None]
            ).reshape(2 * 16 * LAY_CHUNKS, 128)
    bidx = batch_p.reshape(POOL_ROWS, 128)

    ones8 = jnp.ones((128, 8), f32)
    zer8 = jnp.zeros((ROWS_PER_TILE, 8), f32)
    zer32 = jnp.zeros((ROWS_PER_TILE, 32), f32)
    zsum = jnp.zeros((GPAD, HID), f32)
    zcnt = jnp.zeros((GPAD, 8), f32)

    degp = _sc_deg(didx_deg, ones8, zer8)
    dinv, xw0, y0 = _tc_prep0(degp, x_p, W0)

    scat0 = _sc_layer(y0.reshape(2 * NPAD, 32), gidx, didx_lay, zer32)
    y1, xw1 = _tc_mid(scat0, xw0, dinv, b0.reshape(1, HID), W1)
    scat1 = _sc_layer(y1.reshape(2 * NPAD, 32), gidx, didx_lay, zer32)
    y2, xw2 = _tc_mid(scat1, xw1, dinv, b1.reshape(1, HID), W2)
    scat2 = _sc_layer(y2.reshape(2 * NPAD, 32), gidx, didx_lay, zer32)
    h3 = _tc_last(scat2, xw2, dinv, b2.reshape(1, HID))

    sums, cnts = _sc_pool(h3, bidx, ones8, zsum, zcnt)
    cls, reg = _tc_head(sums, cnts, Wc1, bc1.reshape(1, HID // 2), Wc2,
                        bc2.reshape(1, 10), Wr1, br1.reshape(1, HID // 2),
                        Wr2, br2.reshape(1, 1))
    return cls[:G], reg[:G]


# speedup vs baseline: 15.8108x; 1.0032x over previous
import functools

import jax
import jax.numpy as jnp
from jax import lax
from jax.experimental import pallas as pl
from jax.experimental.pallas import tpu as pltpu
from jax.experimental.pallas import tpu_sc as plsc

N = 50000
E = 800000
G = 500
HID = 64

NPAD = 51200
EPAD = 819200
GPAD = 512
TRASH = NPAD - 1

ROWS_PER_TILE = NPAD // 16
DEG_CHUNKS = EPAD // (32 * 128)
LAY_CHUNKS = EPAD // (16 * 128)
POOL_ROWS = 416

_R = 3200
_GRID = NPAD // _R

_SC_MESH = plsc.VectorSubcoreMesh(core_axis_name="c", subcore_axis_name="s")


@functools.partial(
    pl.kernel,
    out_type=jax.ShapeDtypeStruct((2 * NPAD, 8), jnp.float32),
    mesh=_SC_MESH,
    compiler_params=pltpu.CompilerParams(use_tc_tiling_on_sc=False),
    scratch_types=[
        pltpu.VMEM((DEG_CHUNKS, 128), jnp.int32),
        pltpu.VMEM((128, 8), jnp.float32),
        pltpu.VMEM_SHARED((NPAD, 8), jnp.float32),
        pltpu.SemaphoreType.DMA,
    ],
)
def _sc_deg(didx_hbm, ones_hbm, zeros_hbm, out_hbm, didx_v, ones_v, accum,
            dsem):
    c = lax.axis_index("c")
    s = lax.axis_index("s")
    w = s * 2 + c
    stripe = pl.multiple_of(s * ROWS_PER_TILE, 8)
    out_off = pl.multiple_of(c * NPAD + s * ROWS_PER_TILE, 8)
    pltpu.sync_copy(didx_hbm.at[pl.ds(pl.multiple_of(w * DEG_CHUNKS, 8), DEG_CHUNKS)],
                    didx_v)
    pltpu.sync_copy(ones_hbm, ones_v)
    pltpu.sync_copy(zeros_hbm, accum.at[pl.ds(stripe, ROWS_PER_TILE)])
    plsc.subcore_barrier()

    def body(g, carry):
        cps = [
            pltpu.async_copy(ones_v, accum.at[didx_v.at[g * 8 + k]], dsem,
                             add=True)
            for k in range(8)
        ]
        for cp in cps:
            cp.wait()
        return carry

    lax.fori_loop(0, DEG_CHUNKS // 8, body, 0)
    plsc.subcore_barrier()
    pltpu.sync_copy(
        accum.at[pl.ds(stripe, ROWS_PER_TILE)],
        out_hbm.at[pl.ds(out_off, ROWS_PER_TILE)],
    )


_KRING = 4
_IBLK = 40
_IBLK_N = LAY_CHUNKS // _IBLK
_RPB = _IBLK // _KRING


@functools.partial(
    pl.kernel,
    out_type=jax.ShapeDtypeStruct((2 * NPAD, 32), jnp.float32),
    mesh=_SC_MESH,
    compiler_params=pltpu.CompilerParams(use_tc_tiling_on_sc=False),
    scratch_types=[
        pltpu.VMEM((_IBLK, 128), jnp.int32),
        pltpu.VMEM((_IBLK, 128), jnp.int32),
        pltpu.VMEM((_KRING, 128, 32), jnp.float32),
        pltpu.VMEM_SHARED((NPAD, 32), jnp.float32),
        pltpu.SemaphoreType.DMA((_KRING,)),
        pltpu.SemaphoreType.DMA((_KRING,)),
    ],
)
def _sc_layer(yflat_hbm, gidx_hbm, didx_hbm, zeros_hbm, out_hbm,
              gidx_v, didx_v, rows_v, accum, gsem, ssem):
    c = lax.axis_index("c")
    s = lax.axis_index("s")
    stripe = pl.multiple_of(s * ROWS_PER_TILE, 8)
    out_off = pl.multiple_of(c * NPAD + s * ROWS_PER_TILE, 8)
    idx_base = pl.multiple_of((c * 16 + s) * LAY_CHUNKS, 8)
    didx_base = pl.multiple_of(s * LAY_CHUNKS, 8)
    pltpu.sync_copy(zeros_hbm, accum.at[pl.ds(stripe, ROWS_PER_TILE)])
    plsc.subcore_barrier()

    def drain_gather(b):
        pltpu.make_async_copy(yflat_hbm.at[gidx_v.at[0]], rows_v.at[b],
                              gsem.at[b]).wait()

    def drain_scatter(b):
        pltpu.make_async_copy(rows_v.at[b], accum.at[didx_v.at[0]],
                              ssem.at[b]).wait()

    def block_body(blk, carry):
        boff = pl.multiple_of(blk * _IBLK, 8)
        pltpu.sync_copy(gidx_hbm.at[pl.ds(idx_base + boff, _IBLK)], gidx_v)
        pltpu.sync_copy(didx_hbm.at[pl.ds(didx_base + boff, _IBLK)], didx_v)

        def round_body(r, carry2):
            for b in range(_KRING):
                @pl.when(r > 0)
                def _():
                    drain_scatter(b)
                pltpu.async_copy(yflat_hbm.at[gidx_v.at[r * _KRING + b]],
                                 rows_v.at[b], gsem.at[b])
            for b in range(_KRING):
                drain_gather(b)
                pltpu.async_copy(rows_v.at[b],
                                 accum.at[didx_v.at[r * _KRING + b]],
                                 ssem.at[b], add=True)
            return carry2

        lax.fori_loop(0, _RPB, round_body, 0)
        for b in range(_KRING):
            drain_scatter(b)
        return carry

    lax.fori_loop(0, _IBLK_N, block_body, 0)
    plsc.subcore_barrier()
    pltpu.sync_copy(
        accum.at[pl.ds(stripe, ROWS_PER_TILE)],
        out_hbm.at[pl.ds(out_off, ROWS_PER_TILE)],
    )


@functools.partial(
    pl.kernel,
    out_type=(
        jax.ShapeDtypeStruct((2 * GPAD, HID), jnp.float32),
        jax.ShapeDtypeStruct((2 * GPAD, 8), jnp.float32),
    ),
    mesh=_SC_MESH,
    compiler_params=pltpu.CompilerParams(use_tc_tiling_on_sc=False),
    scratch_types=[
        pltpu.VMEM((24, 128), jnp.int32),
        pltpu.VMEM((128, HID), jnp.float32),
        pltpu.VMEM((128, 8), jnp.float32),
        pltpu.VMEM_SHARED((GPAD, HID), jnp.float32),
        pltpu.VMEM_SHARED((GPAD, 8), jnp.float32),
    ],
)
def _sc_pool(h3_hbm, bidx_hbm, ones_hbm, zsum_hbm, zcnt_hbm,
             sums_hbm, cnts_hbm, bidx_v, rows_v, ones_v, accum, cacc):
    c = lax.axis_index("c")
    s = lax.axis_index("s")
    nch = jnp.where(s < 8, 13, 12)
    base = c * 200 + s * 12 + jnp.minimum(s, 8)
    base_al = pl.multiple_of((base // 8) * 8, 8)
    off = base - base_al
    pltpu.sync_copy(bidx_hbm.at[pl.ds(base_al, 24)], bidx_v)
    pltpu.sync_copy(ones_hbm, ones_v)

    @pl.when(s == 0)
    def _():
        pltpu.sync_copy(zsum_hbm, accum)
        pltpu.sync_copy(zcnt_hbm, cacc)

    plsc.subcore_barrier()

    def body(j, carry):
        ch = pl.multiple_of((base + j) * 128, 128)
        pltpu.sync_copy(h3_hbm.at[pl.ds(ch, 128)], rows_v)
        pltpu.sync_copy(rows_v, accum.at[bidx_v.at[off + j]], add=True)
        pltpu.sync_copy(ones_v, cacc.at[bidx_v.at[off + j]], add=True)
        return carry

    lax.fori_loop(0, nch, body, 0)
    plsc.subcore_barrier()
    pltpu.sync_copy(accum.at[pl.ds(pl.multiple_of(s * 32, 8), 32)],
                    sums_hbm.at[pl.ds(pl.multiple_of(c * GPAD + s * 32, 8), 32)])
    pltpu.sync_copy(cacc.at[pl.ds(pl.multiple_of(s * 32, 8), 32)],
                    cnts_hbm.at[pl.ds(pl.multiple_of(c * GPAD + s * 32, 8), 32)])


def _tc_prep0(degp, x, w0):
    def body(d0_ref, d1_ref, x_ref, w0_ref, dinv_ref, xw_ref, y_ref):
        deg = d0_ref[:, 0:1] + d1_ref[:, 0:1] + 1.0
        dinv = lax.rsqrt(deg)
        xw = jnp.dot(x_ref[...], w0_ref[...], preferred_element_type=jnp.float32)
        dinv_ref[...] = jnp.broadcast_to(dinv, (_R, 8))
        xw_ref[...] = xw
        yv = dinv * xw
        y_ref[...] = jnp.concatenate([yv[None, :, :32], yv[None, :, 32:]],
                                     axis=0)

    return pl.pallas_call(
        body,
        grid=(_GRID,),
        in_specs=[
            pl.BlockSpec((_R, 8), lambda i: (i, 0)),
            pl.BlockSpec((_R, 8), lambda i: (i + _GRID, 0)),
            pl.BlockSpec((_R, 3), lambda i: (i, 0)),
            pl.BlockSpec((3, HID), lambda i: (0, 0)),
        ],
        out_specs=[
            pl.BlockSpec((_R, 8), lambda i: (i, 0)),
            pl.BlockSpec((_R, HID), lambda i: (i, 0)),
            pl.BlockSpec((2, _R, 32), lambda i: (0, i, 0)),
        ],
        out_shape=[
            jax.ShapeDtypeStruct((NPAD, 8), jnp.float32),
            jax.ShapeDtypeStruct((NPAD, HID), jnp.float32),
            jax.ShapeDtypeStruct((2, NPAD, 32), jnp.float32),
        ],
    )(degp, degp, x, w0)


def _tc_mid(scat, xwp, dinv, b, w):
    def body(s0_ref, s1_ref, xwp_ref, dinv_ref, b_ref, w_ref, y_ref, xw_ref):
        dv = dinv_ref[:, 0:1]
        sc = jnp.concatenate([s0_ref[...], s1_ref[...]], axis=1)
        h = jnp.maximum(dv * sc + dv * dv * xwp_ref[...] + b_ref[...], 0.0)
        xwn = jnp.dot(h, w_ref[...], preferred_element_type=jnp.float32)
        xw_ref[...] = xwn
        yv = dv * xwn
        y_ref[...] = jnp.concatenate([yv[None, :, :32], yv[None, :, 32:]],
                                     axis=0)

    return pl.pallas_call(
        body,
        grid=(_GRID,),
        in_specs=[
            pl.BlockSpec((_R, 32), lambda i: (i, 0)),
            pl.BlockSpec((_R, 32), lambda i: (i + _GRID, 0)),
            pl.BlockSpec((_R, HID), lambda i: (i, 0)),
            pl.BlockSpec((_R, 8), lambda i: (i, 0)),
            pl.BlockSpec((1, HID), lambda i: (0, 0)),
            pl.BlockSpec((HID, HID), lambda i: (0, 0)),
        ],
        out_specs=[
            pl.BlockSpec((2, _R, 32), lambda i: (0, i, 0)),
            pl.BlockSpec((_R, HID), lambda i: (i, 0)),
        ],
        out_shape=[
            jax.ShapeDtypeStruct((2, NPAD, 32), jnp.float32),
            jax.ShapeDtypeStruct((NPAD, HID), jnp.float32),
        ],
    )(scat, scat, xwp, dinv, b, w)


def _tc_last(scat, xwp, dinv, b):
    def body(s0_ref, s1_ref, xwp_ref, dinv_ref, b_ref, h_ref):
        dv = dinv_ref[:, 0:1]
        sc = jnp.concatenate([s0_ref[...], s1_ref[...]], axis=1)
        h_ref[...] = jnp.maximum(dv * sc + dv * dv * xwp_ref[...] + b_ref[...],
                                 0.0)

    return pl.pallas_call(
        body,
        grid=(_GRID,),
        in_specs=[
            pl.BlockSpec((_R, 32), lambda i: (i, 0)),
            pl.BlockSpec((_R, 32), lambda i: (i + _GRID, 0)),
            pl.BlockSpec((_R, HID), lambda i: (i, 0)),
            pl.BlockSpec((_R, 8), lambda i: (i, 0)),
            pl.BlockSpec((1, HID), lambda i: (0, 0)),
        ],
        out_specs=pl.BlockSpec((_R, HID), lambda i: (i, 0)),
        out_shape=jax.ShapeDtypeStruct((NPAD, HID), jnp.float32),
    )(scat, scat, xwp, dinv, b)


def _tc_head(sums, cnts, wc1, bc1, wc2, bc2, wr1, br1, wr2, br2):
    def body(p_ref, c_ref, wc1_ref, bc1_ref, wc2_ref, bc2_ref,
             wr1_ref, br1_ref, wr2_ref, br2_ref, cls_ref, reg_ref):
        ssum = p_ref[0:GPAD, :] + p_ref[GPAD:2 * GPAD, :]
        cnt = c_ref[0:GPAD, 0:1] + c_ref[GPAD:2 * GPAD, 0:1]
        pooled = ssum / jnp.maximum(cnt, 1.0)
        hc = jnp.maximum(
            jnp.dot(pooled, wc1_ref[...], preferred_element_type=jnp.float32)
            + bc1_ref[...], 0.0)
        cls_ref[...] = (jnp.dot(hc, wc2_ref[...], preferred_element_type=jnp.float32)
                        + bc2_ref[...])
        hr = jnp.maximum(
            jnp.dot(pooled, wr1_ref[...], preferred_element_type=jnp.float32)
            + br1_ref[...], 0.0)
        reg_ref[...] = (jnp.dot(hr, wr2_ref[...], preferred_element_type=jnp.float32)
                        + br2_ref[...])

    return pl.pallas_call(
        body,
        out_shape=[
            jax.ShapeDtypeStruct((GPAD, 10), jnp.float32),
            jax.ShapeDtypeStruct((GPAD, 1), jnp.float32),
        ],
    )(sums, cnts, wc1, bc1, wc2, bc2, wr1, br1, wr2, br2)


def kernel(x, W0, b0, W1, b1, W2, b2, Wc1, bc1, Wc2, bc2, Wr1, br1, Wr2, br2,
           edge_index, batch):
    f32 = jnp.float32
    src = edge_index[0]
    dst = edge_index[1]
    pad_e = jnp.full((EPAD - E,), TRASH, jnp.int32)
    src_p = jnp.concatenate([src, pad_e])
    dst_p = jnp.concatenate([dst, pad_e])
    x_p = jnp.pad(x, ((0, NPAD - N), (0, 0)))
    batch_p = jnp.concatenate(
        [batch, jnp.full((POOL_ROWS * 128 - N,), GPAD - 1, jnp.int32)])

    didx = dst_p.reshape(16 * LAY_CHUNKS, 128)
    gidx = (src_p[None, :] + jnp.array([0, NPAD], jnp.int32)[:, None]
            ).reshape(2 * 16 * LAY_CHUNKS, 128)
    bidx = batch_p.reshape(POOL_ROWS, 128)

    ones8 = jnp.ones((128, 8), f32)
    zer8 = jnp.zeros((ROWS_PER_TILE, 8), f32)
    zer32 = jnp.zeros((ROWS_PER_TILE, 32), f32)
    zsum = jnp.zeros((GPAD, HID), f32)
    zcnt = jnp.zeros((GPAD, 8), f32)

    degp = _sc_deg(didx, ones8, zer8)
    dinv, xw0, y0 = _tc_prep0(degp, x_p, W0)

    scat0 = _sc_layer(y0.reshape(2 * NPAD, 32), gidx, didx, zer32)
    y1, xw1 = _tc_mid(scat0, xw0, dinv, b0.reshape(1, HID), W1)
    scat1 = _sc_layer(y1.reshape(2 * NPAD, 32), gidx, didx, zer32)
    y2, xw2 = _tc_mid(scat1, xw1, dinv, b1.reshape(1, HID), W2)
    scat2 = _sc_layer(y2.reshape(2 * NPAD, 32), gidx, didx, zer32)
    h3 = _tc_last(scat2, xw2, dinv, b2.reshape(1, HID))

    sums, cnts = _sc_pool(h3, bidx, ones8, zsum, zcnt)
    cls, reg = _tc_head(sums, cnts, Wc1, bc1.reshape(1, HID // 2), Wc2,
                        bc2.reshape(1, 10), Wr1, br1.reshape(1, HID // 2),
                        Wr2, br2.reshape(1, 1))
    return cls[:G], reg[:G]
